# baseline (speedup 1.0000x reference)
import functools

import jax
import jax.numpy as jnp
from jax import lax
from jax.experimental import pallas as pl
from jax.experimental.pallas import tpu as pltpu
from jax.experimental.pallas import tpu_sc as plsc

SZ = 4096
TEMP = 10.0
EPS = 1e-20
LN2 = 0.6931471805599453
SQRT2 = 1.4142135623730951

NC = 2
NS = 16
NW = NC * NS
LANES = 16

N_OUT = SZ * SZ
PER_W = N_OUT // NW
C_OUT = 16384
C_IN = 2 * C_OUT
CHUNKS = PER_W // C_OUT

_C0 = 1.4426991769054545
_C1 = -0.7212366511576747
_C2 = 0.4800737469155951
_C3 = -0.36592988270923904
_C4 = 0.31470880562262726
_C5 = -0.20438587444643186


def _log2(x):
    xi = plsc.bitcast(x, jnp.int32)
    e = (xi >> 23) - 127
    m = plsc.bitcast((xi & 0x007FFFFF) | 0x3F800000, jnp.float32)
    big = m >= SQRT2
    e = jnp.where(big, e + 1, e)
    m = jnp.where(big, m * 0.5, m)
    t = m - 1.0
    q = _C5
    q = q * t + _C4
    q = q * t + _C3
    q = q * t + _C2
    q = q * t + _C1
    q = q * t + _C0
    return e.astype(jnp.float32) + t * q


@functools.partial(
    pl.kernel,
    out_type=jax.ShapeDtypeStruct((N_OUT,), jnp.float32),
    mesh=plsc.VectorSubcoreMesh(
        core_axis_name="c", subcore_axis_name="s", num_cores=NC, num_subcores=NS
    ),
    scratch_types=[
        pltpu.VMEM((C_IN,), jnp.float32),
        pltpu.VMEM((C_IN,), jnp.float32),
        pltpu.VMEM((C_OUT,), jnp.float32),
    ],
    compiler_params=pltpu.CompilerParams(needs_layout_passes=False),
)
def _gumbel_sc(gm_hbm, u_hbm, out_hbm, gm_v, u_v, o_v):
    wid = lax.axis_index("s") * NC + lax.axis_index("c")
    in_base = wid * (PER_W * 2)
    out_base = wid * PER_W

    iota = lax.iota(jnp.int32, LANES)

    def chunk_body(g, _):
        in_off = in_base + g * C_IN
        out_off = out_base + g * C_OUT
        pltpu.sync_copy(gm_hbm.at[pl.ds(in_off, C_IN)], gm_v)
        pltpu.sync_copy(u_hbm.at[pl.ds(in_off, C_IN)], u_v)

        def inner(j, _):
            idx_e = j * 32 + 2 * iota
            idx_o = idx_e + 1
            ge = plsc.load_gather(gm_v, [idx_e])
            go = plsc.load_gather(gm_v, [idx_o])
            ue = plsc.load_gather(u_v, [idx_e])
            uo = plsc.load_gather(u_v, [idx_o])
            se = _log2(EPS - LN2 * _log2(ue + EPS))
            so = _log2(EPS - LN2 * _log2(uo + EPS))
            darg = ((go - ge) - LN2 * (so - se)) * (1.0 / TEMP)
            o_v[pl.ds(j * LANES, LANES)] = 1.0 / (1.0 + jnp.exp(darg))
            return 0

        lax.fori_loop(0, C_OUT // LANES, inner, 0)
        pltpu.sync_copy(o_v, out_hbm.at[pl.ds(out_off, C_OUT)])
        return 0

    lax.fori_loop(0, CHUNKS, chunk_body, 0)


def kernel(gen_matrix, u):
    out = _gumbel_sc(gen_matrix.reshape(-1), u.reshape(-1))
    return out.reshape(SZ, SZ)


# speedup vs baseline: 183.8810x; 183.8810x over previous
import jax
import jax.numpy as jnp
from jax.experimental import pallas as pl

SZ = 4096
TEMP = 10.0
EPS = 1e-20

RI = 16


def _body(gm_ref, u_ref, o_ref):
    ge = gm_ref[:, 0, :]
    go = gm_ref[:, 1, :]
    ue = u_ref[:, 0, :]
    uo = u_ref[:, 1, :]
    gbe = -jnp.log(-jnp.log(ue + EPS) + EPS)
    gbo = -jnp.log(-jnp.log(uo + EPS) + EPS)
    darg = ((go + gbo) - (ge + gbe)) * (1.0 / TEMP)
    o_ref[...] = 1.0 / (1.0 + jnp.exp(darg))


def kernel(gen_matrix, u):
    gmt = gen_matrix.swapaxes(1, 2)
    ut = u.reshape(SZ, SZ, 2).swapaxes(1, 2)
    return pl.pallas_call(
        _body,
        grid=(SZ // RI,),
        in_specs=[
            pl.BlockSpec((RI, 2, SZ), lambda g: (g, 0, 0)),
            pl.BlockSpec((RI, 2, SZ), lambda g: (g, 0, 0)),
        ],
        out_specs=pl.BlockSpec((RI, SZ), lambda g: (g, 0)),
        out_shape=jax.ShapeDtypeStruct((SZ, SZ), jnp.float32),
    )(gmt, ut)
